# baseline (speedup 1.0000x reference)
import jax
import jax.numpy as jnp
from jax import lax
from jax.experimental import pallas as pl
from jax.experimental.pallas import tpu as pltpu
from jax.experimental.pallas import tpu_sc as plsc

BATCH = 1024
FDIM = 128
NPOS = 225
PPOS = 256
PCODE = 2380
EMB = 2 * PCODE + 2
PCPAD = 4768
NEXT = 40
SEBASE = PCPAD
ZROW = PCPAD + 36
HALF = 128
NRING = 3
NW = 32
SPW = BATCH // NW


def _sc_body(pk_hbm, offm_hbm, pcode_hbm, symb_hbm, out_hbm,
             pk_v, off_v, idx_v, idxcp_v, poscp_v, rows_v, trans_v,
             pcode_sp, sem0, sem1, sem2, sem_st, sem_out, sem_pk):
    sems = (sem0, sem1, sem2)
    cid = lax.axis_index("c")
    sid = lax.axis_index("s")
    wid = sid * 2 + cid
    iota = lax.iota(jnp.int32, 16)
    rowbase = iota * NPOS
    fzero = jnp.zeros((16,), jnp.float32)

    pltpu.sync_copy(offm_hbm, off_v)
    pltpu.async_copy(pk_hbm.at[wid * SPW], pk_v.at[0], sem_pk)

    @pl.when(sid == 0)
    def _stage():
        pltpu.async_copy(pcode_hbm, pcode_sp.at[pl.ds(0, PCPAD)],
                         sem_st).wait()
        for m in range(36):
            pltpu.async_copy(symb_hbm.at[PCODE + m * EMB],
                             rows_v.at[1, 2 * m], sem_st)
            pltpu.async_copy(symb_hbm.at[2 * PCODE + 1 + m * EMB],
                             rows_v.at[1, 2 * m + 1], sem_st)
        pltpu.make_async_copy(symb_hbm.at[pl.ds(0, 72)],
                              rows_v.at[1].at[pl.ds(0, 72)], sem_st).wait()
        pltpu.sync_copy(pcode_sp.at[PCODE], rows_v.at[1, 72])
        pltpu.sync_copy(pcode_sp.at[2 * PCODE + 1], rows_v.at[1, 73])
        def _pair(m, c):
            for k in range(8):
                sl = pl.ds(16 * k, 16)
                rows_v[0, m, sl] = (rows_v[1, 2 * m, sl]
                                    + rows_v[1, 2 * m + 1, sl]
                                    + rows_v[1, 72, sl] + rows_v[1, 73, sl])
            return c
        lax.fori_loop(0, 36, _pair, 0)
        for z in range(36, NEXT):
            for k in range(8):
                rows_v[0, z, pl.ds(16 * k, 16)] = fzero
        pltpu.sync_copy(rows_v.at[0].at[pl.ds(0, NEXT)],
                        pcode_sp.at[pl.ds(PCPAD, NEXT)])
    plsc.subcore_barrier()

    def sample_body(i, carry):
        b = wid * SPW + i
        cur = i % 2
        pltpu.make_async_copy(pk_hbm.at[b], pk_v.at[cur], sem_pk).wait()
        nxt = jnp.minimum(b + 1, BATCH - 1)
        pltpu.async_copy(pk_hbm.at[nxt], pk_v.at[1 - cur], sem_pk)

        for q in range(4):
            for l in range(8):
                idxcp_v[q, pl.ds(16 * l, 16)] = iota + 16 * l

        cnts = [jnp.int32(0)] * 4
        for t in range(16):
            sl = pl.ds(16 * t, 16)
            h, loc = t // 8, 16 * (t % 8)
            dsl = pl.ds(loc, 16)
            w = pk_v[cur, sl]
            ne = (w >> 24) > 0
            em = jnp.logical_not(ne)
            off = off_v[sl]
            mv = off // EMB
            s0 = w & 0xFFF
            s1 = ((w >> 12) & 0xFFF) + (PCODE + 1)
            idx_v[0 + h, dsl] = jnp.where(ne, SEBASE + mv, s0)
            idx_v[2 + h, dsl] = jnp.where(ne, ZROW, s1)
            if t == 14:
                em = em & (iota < 1)
            if t == 15:
                continue
            for ch, val in ((0, s0 + off), (1, s1 + off)):
                q = 2 * ch + h
                cnt = cnts[q]
                plsc.store_compressed(idxcp_v.at[q, pl.ds(cnt, 16)], val, mask=em)
                plsc.store_compressed(poscp_v.at[q, pl.ds(cnt, 16)],
                                      iota + loc, mask=em)
                cnts[q] = cnt + plsc.all_reduce_population_count(em)[0]

        sched = (("s", 0, "store", 0), ("s", 1, "store", 1),
                 ("q", 0, "add", 0), ("q", 1, "add", 1),
                 ("q", 2, "add", 0), ("q", 3, "add", 1),
                 ("s", 2, "add", 0), ("s", 3, "add", 1))

        def ngroups(q):
            return (cnts[q] + 15) >> 4

        def fire(ci):
            kind, a, _, _ = sched[ci]
            buf = ci % NRING
            if kind == "s":
                pltpu.async_copy(pcode_sp.at[idx_v.at[a]], rows_v.at[buf],
                                 sems[buf])
            else:
                def issue(g, c2, a=a, buf=buf):
                    vec = idxcp_v[a, pl.ds(16 * g, 16)]
                    for r in range(16):
                        pltpu.async_copy(symb_hbm.at[vec[r]],
                                         rows_v.at[buf, 16 * g + r],
                                         sems[buf])
                    return c2
                lax.fori_loop(0, ngroups(a), issue, 0)

        def drain(ci):
            kind, a, _, _ = sched[ci]
            buf = ci % NRING
            if kind == "s":
                pltpu.make_async_copy(symb_hbm.at[pl.ds(0, HALF)],
                                      rows_v.at[buf], sems[buf]).wait()
            else:
                def dwait(g, c2, buf=buf):
                    pltpu.make_async_copy(symb_hbm.at[pl.ds(0, 16)],
                                          rows_v.at[buf].at[pl.ds(0, 16)],
                                          sems[buf]).wait()
                    return c2
                lax.fori_loop(0, ngroups(a), dwait, 0)

        def scatter(ci):
            kind, a, mode, h = sched[ci]
            buf = ci % NRING
            base_col = HALF * h
            if kind == "s":
                cmax = HALF if h == 0 else NPOS - HALF

                @plsc.parallel_loop(0, cmax, unroll=8)
                def _cb(c, buf=buf, first=(mode == "store"), base_col=base_col):
                    for k in range(8):
                        v = rows_v[buf, c, pl.ds(16 * k, 16)]
                        fidx = rowbase + (16 * k * NPOS + base_col + c)
                        if first:
                            plsc.store_scatter(trans_v, [fidx], v)
                        else:
                            plsc.addupdate_scatter(trans_v, [fidx], v)
            else:
                cnt = cnts[a]

                def gbody(g, c2, a=a, buf=buf, base_col=base_col, cnt=cnt):
                    posv = poscp_v[a, pl.ds(16 * g, 16)]
                    for r in range(16):
                        c = 16 * g + r

                        @pl.when(c < cnt)
                        def _one(c=c, r=r, posv=posv, buf=buf,
                                 base_col=base_col):
                            col = posv[r] + base_col
                            for k in range(8):
                                v = rows_v[buf, c, pl.ds(16 * k, 16)]
                                fidx = rowbase + 16 * k * NPOS + col
                                plsc.addupdate_scatter(trans_v, [fidx], v)
                    return c2
                lax.fori_loop(0, ngroups(a), gbody, 0)

        for ci in range(NRING):
            fire(ci)
        @pl.when(i > 0)
        def _wout():
            pltpu.make_async_copy(trans_v, out_hbm.at[b], sem_out).wait()
        for ci in range(len(sched)):
            drain(ci)
            scatter(ci)
            if ci + NRING < len(sched):
                fire(ci + NRING)

        pltpu.async_copy(trans_v, out_hbm.at[b], sem_out)
        return carry

    lax.fori_loop(0, SPW, sample_body, 0)
    pltpu.make_async_copy(trans_v, out_hbm.at[wid * SPW], sem_out).wait()
    pltpu.make_async_copy(pk_hbm.at[0], pk_v.at[0], sem_pk).wait()


def kernel(sparse_feature_dim, sparse_feature_input, board_input,
           pcode_table, symboard_table, offset_map):
    del sparse_feature_dim
    sfi = sparse_feature_input[:, 10:12].reshape(BATCH, 2, NPOS)
    sfi = jnp.pad(sfi, ((0, 0), (0, 0), (0, PPOS - NPOS)))
    brd = board_input.reshape(BATCH, 2, NPOS)
    brd = jnp.pad(brd, ((0, 0), (0, 0), (0, PPOS - NPOS)))
    pk = (sfi[:, 0] | (sfi[:, 1] << 12) | (brd[:, 0] << 24)
          | (brd[:, 1] << 25))
    offm = jnp.pad(offset_map.reshape(NPOS), (0, PPOS - NPOS))
    pcode_pad = jnp.pad(pcode_table, ((0, PCPAD - EMB), (0, 0)))

    mesh = plsc.VectorSubcoreMesh(core_axis_name="c", subcore_axis_name="s")
    run = pl.kernel(
        _sc_body, mesh=mesh,
        compiler_params=pltpu.CompilerParams(needs_layout_passes=False),
        out_type=jax.ShapeDtypeStruct((BATCH, FDIM * NPOS), jnp.float32),
        scratch_types=[
            pltpu.VMEM((2, PPOS), jnp.int32),
            pltpu.VMEM((PPOS,), jnp.int32),
            pltpu.VMEM((4, HALF), jnp.int32),
            pltpu.VMEM((4, HALF), jnp.int32),
            pltpu.VMEM((4, HALF), jnp.int32),
            pltpu.VMEM((NRING, HALF, FDIM), jnp.float32),
            pltpu.VMEM((FDIM * NPOS,), jnp.float32),
            pltpu.VMEM_SHARED((PCPAD + NEXT, FDIM), jnp.float32),
            pltpu.SemaphoreType.DMA,
            pltpu.SemaphoreType.DMA,
            pltpu.SemaphoreType.DMA,
            pltpu.SemaphoreType.DMA,
            pltpu.SemaphoreType.DMA,
            pltpu.SemaphoreType.DMA,
        ],
    )
    out = run(pk, offm, pcode_pad, symboard_table)
    return out.reshape(BATCH, FDIM, 15, 15)


# speedup vs baseline: 1.0214x; 1.0214x over previous
import jax
import jax.numpy as jnp
from jax import lax
from jax.experimental import pallas as pl
from jax.experimental.pallas import tpu as pltpu
from jax.experimental.pallas import tpu_sc as plsc

BATCH = 1024
FDIM = 128
NPOS = 225
PPOS = 256
PCODE = 2380
EMB = 2 * PCODE + 2
PCPAD = 4768
NEXT = 40
SEBASE = PCPAD
ZROW = PCPAD + 36
HALF = 128
NRING = 3
NW = 32
SPW = BATCH // NW


def _sc_body(pk_hbm, offm_hbm, pcode_hbm, symb_hbm, out_hbm,
             pk_v, off_v, idx_v, idxcp_v, poscp_v, rows_v, trans_v,
             pcode_sp, sem0, sem1, sem2, sem_st, sem_out, sem_pk):
    sems = (sem0, sem1, sem2)
    cid = lax.axis_index("c")
    sid = lax.axis_index("s")
    wid = sid * 2 + cid
    iota = lax.iota(jnp.int32, 16)
    rowbase = iota * NPOS
    fzero = jnp.zeros((16,), jnp.float32)

    pltpu.sync_copy(offm_hbm, off_v)
    pltpu.async_copy(pk_hbm.at[wid * SPW], pk_v.at[0], sem_pk)

    @pl.when(sid == 0)
    def _stage():
        pltpu.async_copy(pcode_hbm, pcode_sp.at[pl.ds(0, PCPAD)],
                         sem_st).wait()
        for m in range(36):
            pltpu.async_copy(symb_hbm.at[PCODE + m * EMB],
                             rows_v.at[1, 2 * m], sem_st)
            pltpu.async_copy(symb_hbm.at[2 * PCODE + 1 + m * EMB],
                             rows_v.at[1, 2 * m + 1], sem_st)
        pltpu.make_async_copy(symb_hbm.at[pl.ds(0, 72)],
                              rows_v.at[1].at[pl.ds(0, 72)], sem_st).wait()
        pltpu.sync_copy(pcode_sp.at[PCODE], rows_v.at[1, 72])
        pltpu.sync_copy(pcode_sp.at[2 * PCODE + 1], rows_v.at[1, 73])
        def _pair(m, c):
            for k in range(8):
                sl = pl.ds(16 * k, 16)
                rows_v[0, m, sl] = (rows_v[1, 2 * m, sl]
                                    + rows_v[1, 2 * m + 1, sl]
                                    + rows_v[1, 72, sl] + rows_v[1, 73, sl])
            return c
        lax.fori_loop(0, 36, _pair, 0)
        for z in range(36, NEXT):
            for k in range(8):
                rows_v[0, z, pl.ds(16 * k, 16)] = fzero
        pltpu.sync_copy(rows_v.at[0].at[pl.ds(0, NEXT)],
                        pcode_sp.at[pl.ds(PCPAD, NEXT)])
    plsc.subcore_barrier()

    def sample_body(i, carry):
        b = wid * SPW + i
        cur = i % 2
        pltpu.make_async_copy(pk_hbm.at[b], pk_v.at[cur], sem_pk).wait()
        nxt = jnp.minimum(b + 1, BATCH - 1)
        pltpu.async_copy(pk_hbm.at[nxt], pk_v.at[1 - cur], sem_pk)

        for q in range(4):
            for l in range(8):
                idxcp_v[q, pl.ds(16 * l, 16)] = iota + 16 * l

        cnts = [jnp.int32(0)] * 4
        for t in range(16):
            sl = pl.ds(16 * t, 16)
            h, loc = t // 8, 16 * (t % 8)
            dsl = pl.ds(loc, 16)
            w = pk_v[cur, sl]
            ne = (w >> 24) > 0
            em = jnp.logical_not(ne)
            off = off_v[sl]
            mv = off // EMB
            s0 = w & 0xFFF
            s1 = ((w >> 12) & 0xFFF) + (PCODE + 1)
            idx_v[0 + h, dsl] = jnp.where(ne, SEBASE + mv, s0)
            idx_v[2 + h, dsl] = jnp.where(ne, ZROW, s1)
            if t == 14:
                em = em & (iota < 1)
            if t == 15:
                continue
            for ch, val in ((0, s0 + off), (1, s1 + off)):
                q = 2 * ch + h
                cnt = cnts[q]
                plsc.store_compressed(idxcp_v.at[q, pl.ds(cnt, 16)], val, mask=em)
                plsc.store_compressed(poscp_v.at[q, pl.ds(cnt, 16)],
                                      iota + loc, mask=em)
                cnts[q] = cnt + plsc.all_reduce_population_count(em)[0]

        sched = (("s", 0, "store", 0), ("s", 1, "store", 1),
                 ("q", 0, "add", 0), ("q", 1, "add", 1),
                 ("q", 2, "add", 0), ("q", 3, "add", 1),
                 ("s", 2, "add", 0), ("s", 3, "add", 1))

        def ngroups(q):
            return (cnts[q] + 15) >> 4

        def fire(ci):
            kind, a, _, _ = sched[ci]
            buf = ci % NRING
            if kind == "s":
                pltpu.async_copy(pcode_sp.at[idx_v.at[a]], rows_v.at[buf],
                                 sems[buf])
            else:
                def issue(g, c2, a=a, buf=buf):
                    vec = idxcp_v[a, pl.ds(16 * g, 16)]
                    for r in range(16):
                        pltpu.async_copy(symb_hbm.at[vec[r]],
                                         rows_v.at[buf, 16 * g + r],
                                         sems[buf])
                    return c2
                lax.fori_loop(0, ngroups(a), issue, 0)

        def drain(ci):
            kind, a, _, _ = sched[ci]
            buf = ci % NRING
            if kind == "s":
                pltpu.make_async_copy(symb_hbm.at[pl.ds(0, HALF)],
                                      rows_v.at[buf], sems[buf]).wait()
            else:
                def dwait(g, c2, buf=buf):
                    pltpu.make_async_copy(symb_hbm.at[pl.ds(0, 16)],
                                          rows_v.at[buf].at[pl.ds(0, 16)],
                                          sems[buf]).wait()
                    return c2
                lax.fori_loop(0, ngroups(a), dwait, 0)

        def scatter(ci):
            kind, a, mode, h = sched[ci]
            buf = ci % NRING
            base_col = HALF * h
            if kind == "s":
                cmax = HALF if h == 0 else NPOS - HALF

                @plsc.parallel_loop(0, cmax, unroll=4)
                def _cb(c, buf=buf, first=(mode == "store"), base_col=base_col):
                    for k in range(8):
                        v = rows_v[buf, c, pl.ds(16 * k, 16)]
                        fidx = rowbase + (16 * k * NPOS + base_col + c)
                        if first:
                            plsc.store_scatter(trans_v, [fidx], v)
                        else:
                            plsc.addupdate_scatter(trans_v, [fidx], v)
            else:
                cnt = cnts[a]

                def gbody(g, c2, a=a, buf=buf, base_col=base_col, cnt=cnt):
                    posv = poscp_v[a, pl.ds(16 * g, 16)]
                    for r in range(16):
                        c = 16 * g + r

                        @pl.when(c < cnt)
                        def _one(c=c, r=r, posv=posv, buf=buf,
                                 base_col=base_col):
                            col = posv[r] + base_col
                            for k in range(8):
                                v = rows_v[buf, c, pl.ds(16 * k, 16)]
                                fidx = rowbase + 16 * k * NPOS + col
                                plsc.addupdate_scatter(trans_v, [fidx], v)
                    return c2
                lax.fori_loop(0, ngroups(a), gbody, 0)

        for ci in range(NRING):
            fire(ci)
        @pl.when(i > 0)
        def _wout():
            pltpu.make_async_copy(trans_v, out_hbm.at[b], sem_out).wait()
        for ci in range(len(sched)):
            drain(ci)
            scatter(ci)
            if ci + NRING < len(sched):
                fire(ci + NRING)

        pltpu.async_copy(trans_v, out_hbm.at[b], sem_out)
        return carry

    lax.fori_loop(0, SPW, sample_body, 0)
    pltpu.make_async_copy(trans_v, out_hbm.at[wid * SPW], sem_out).wait()
    pltpu.make_async_copy(pk_hbm.at[0], pk_v.at[0], sem_pk).wait()


def kernel(sparse_feature_dim, sparse_feature_input, board_input,
           pcode_table, symboard_table, offset_map):
    del sparse_feature_dim
    sfi = sparse_feature_input[:, 10:12].reshape(BATCH, 2, NPOS)
    sfi = jnp.pad(sfi, ((0, 0), (0, 0), (0, PPOS - NPOS)))
    brd = board_input.reshape(BATCH, 2, NPOS)
    brd = jnp.pad(brd, ((0, 0), (0, 0), (0, PPOS - NPOS)))
    pk = (sfi[:, 0] | (sfi[:, 1] << 12) | (brd[:, 0] << 24)
          | (brd[:, 1] << 25))
    offm = jnp.pad(offset_map.reshape(NPOS), (0, PPOS - NPOS))
    pcode_pad = jnp.pad(pcode_table, ((0, PCPAD - EMB), (0, 0)))

    mesh = plsc.VectorSubcoreMesh(core_axis_name="c", subcore_axis_name="s")
    run = pl.kernel(
        _sc_body, mesh=mesh,
        compiler_params=pltpu.CompilerParams(needs_layout_passes=False),
        out_type=jax.ShapeDtypeStruct((BATCH, FDIM * NPOS), jnp.float32),
        scratch_types=[
            pltpu.VMEM((2, PPOS), jnp.int32),
            pltpu.VMEM((PPOS,), jnp.int32),
            pltpu.VMEM((4, HALF), jnp.int32),
            pltpu.VMEM((4, HALF), jnp.int32),
            pltpu.VMEM((4, HALF), jnp.int32),
            pltpu.VMEM((NRING, HALF, FDIM), jnp.float32),
            pltpu.VMEM((FDIM * NPOS,), jnp.float32),
            pltpu.VMEM_SHARED((PCPAD + NEXT, FDIM), jnp.float32),
            pltpu.SemaphoreType.DMA,
            pltpu.SemaphoreType.DMA,
            pltpu.SemaphoreType.DMA,
            pltpu.SemaphoreType.DMA,
            pltpu.SemaphoreType.DMA,
            pltpu.SemaphoreType.DMA,
        ],
    )
    out = run(pk, offm, pcode_pad, symboard_table)
    return out.reshape(BATCH, FDIM, 15, 15)
